# baseline (speedup 1.0000x reference)
import functools

import jax
import jax.numpy as jnp
from jax import lax
from jax.experimental import pallas as pl
from jax.experimental.pallas import tpu as pltpu
from jax.experimental.pallas import tpu_sc as plsc

_PERCENT = 0.33


def _score_body(x_ref, oh_ref, wt_ref, b_ref, y_ref, score_ref):
    C, HW = x_ref.shape[1], x_ref.shape[2]
    xb = x_ref[0]
    oh = oh_ref[0]
    wt = wt_ref[...]
    wl_col = jnp.sum(wt * oh, axis=1, keepdims=True)
    pooled = jnp.sum(xb, axis=1, keepdims=True) * (1.0 / HW)
    y_ref[0] = jnp.sum(wt * pooled, axis=0, keepdims=True) + b_ref[...]
    s = xb * wl_col
    mx = jnp.max(s, axis=0, keepdims=True)
    mn = jnp.min(s, axis=0, keepdims=True)
    den = mx - mn
    r = 1.0 / den
    r = r * (2.0 - den * r)
    r = r * (2.0 - den * r)
    sn = (s - mn) * r
    score_ref[0] = jnp.sum(sn, axis=1, keepdims=True)


def _make_sc_mask(B, C, drop):
    info = plsc.get_sparse_core_info()
    nw = info.num_cores * info.num_subcores
    rows_per = B // nw
    nslice = C // 16
    mesh = plsc.VectorSubcoreMesh(core_axis_name="c", subcore_axis_name="s")

    @functools.partial(
        pl.kernel, mesh=mesh,
        out_type=jax.ShapeDtypeStruct((B, C), jnp.float32),
        scratch_types=[
            pltpu.VMEM((rows_per, C), jnp.float32),
            pltpu.VMEM((rows_per, C), jnp.int32),
            pltpu.VMEM((rows_per, C), jnp.float32),
        ],
    )
    def mask_kernel(scores_hbm, out_hbm, rows_v, keys_v, mask_v):
        wid = lax.axis_index("s") * info.num_cores + lax.axis_index("c")
        base = wid * rows_per
        pltpu.sync_copy(scores_hbm.at[pl.ds(base, rows_per)], rows_v)
        lane = lax.iota(jnp.int32, 16)

        def _take16(x, idx):
            return lax.gather(
                x, idx[:, None],
                lax.GatherDimensionNumbers(
                    offset_dims=(), collapsed_slice_dims=(0,),
                    start_index_map=(0,)),
                (1,), mode=lax.GatherScatterMode.PROMISE_IN_BOUNDS)

        def _prefix16(v):
            for k in (1, 2, 4, 8):
                sh = _take16(v, jnp.maximum(lane - k, 0))
                v = v + jnp.where(lane >= k, sh, jnp.int32(0))
            return v

        def _splat_last(v):
            return _take16(v, jnp.full((16,), jnp.int32(15)))

        for r in range(rows_per):
            for j in range(nslice):
                v = rows_v[r, pl.ds(16 * j, 16)]
                v = jnp.where(v == 0.0, 0.0, v)
                iv = lax.bitcast_convert_type(v, jnp.int32)
                keys_v[r, pl.ds(16 * j, 16)] = (
                    iv ^ ((iv >> 31) & jnp.int32(0x7FFFFFFF)))

            lo0 = jnp.full((16,), jnp.int32(-2147483648))
            hi0 = jnp.full((16,), jnp.int32(2147483647))

            def bis(_, carry):
                lo, hi = carry
                mid = (lo >> 1) + (hi >> 1) + ((lo | hi) & 1)
                cntv = jnp.zeros((16,), jnp.int32)
                for j in range(nslice):
                    k16 = keys_v[r, pl.ds(16 * j, 16)]
                    cntv = cntv + jnp.where(k16 >= mid, jnp.int32(1),
                                            jnp.int32(0))
                cnt = _splat_last(_prefix16(cntv))
                ge = cnt >= drop
                return (jnp.where(ge, mid, lo), jnp.where(ge, hi, mid - 1))

            tau, _ = lax.fori_loop(0, 32, bis, (lo0, hi0))

            gtv = jnp.zeros((16,), jnp.int32)
            for j in range(nslice):
                k16 = keys_v[r, pl.ds(16 * j, 16)]
                gtv = gtv + jnp.where(k16 > tau, jnp.int32(1), jnp.int32(0))
            quota = drop - _splat_last(_prefix16(gtv))
            run = jnp.zeros((16,), jnp.int32)
            for j in range(nslice):
                k16 = keys_v[r, pl.ds(16 * j, 16)]
                gt = k16 > tau
                eq = k16 == tau
                eqi = jnp.where(eq, jnp.int32(1), jnp.int32(0))
                pre = _prefix16(eqi)
                cum = run + pre
                dropped = jnp.logical_or(
                    gt, jnp.logical_and(eq, cum <= quota))
                mask_v[r, pl.ds(16 * j, 16)] = jnp.where(dropped, 0.0, 1.0)
                run = run + _splat_last(pre)
        pltpu.sync_copy(mask_v, out_hbm.at[pl.ds(base, rows_per)])

    return mask_kernel


def kernel(x, labels, W, b):
    B, C, H, Wd = x.shape
    K = W.shape[0]
    HW = H * Wd
    drop = int(C * _PERCENT)
    x3 = x.reshape(B, C, HW)
    oh = (labels.astype(jnp.int32)[:, None]
          == jnp.arange(K, dtype=jnp.int32)[None, :]).astype(jnp.float32)
    oh3 = oh.reshape(B, 1, K)
    b2 = b.reshape(1, K).astype(jnp.float32)
    y, score = pl.pallas_call(
        _score_body,
        grid=(B,),
        in_specs=[
            pl.BlockSpec((1, C, HW), lambda i: (i, 0, 0)),
            pl.BlockSpec((1, 1, K), lambda i: (i, 0, 0)),
            pl.BlockSpec((C, K), lambda i: (0, 0)),
            pl.BlockSpec((1, K), lambda i: (0, 0)),
        ],
        out_specs=(
            pl.BlockSpec((1, 1, K), lambda i: (i, 0, 0)),
            pl.BlockSpec((1, C, 1), lambda i: (i, 0, 0)),
        ),
        out_shape=(
            jax.ShapeDtypeStruct((B, 1, K), jnp.float32),
            jax.ShapeDtypeStruct((B, C, 1), jnp.float32),
        ),
        compiler_params=pltpu.CompilerParams(
            dimension_semantics=("parallel",),
        ),
    )(x3, oh3, W.T, b2)
    mask = _make_sc_mask(B, C, drop)(score.reshape(B, C))
    return (y.reshape(B, K), mask.reshape(B, C, 1, 1))


# speedup vs baseline: 1.0073x; 1.0073x over previous
import functools

import jax
import jax.numpy as jnp
from jax import lax
from jax.experimental import pallas as pl
from jax.experimental.pallas import tpu as pltpu
from jax.experimental.pallas import tpu_sc as plsc

_PERCENT = 0.33


def _score_body(x_ref, oh_ref, wt_ref, b_ref, y_ref, score_ref):
    C, HW = x_ref.shape[1], x_ref.shape[2]
    xb = x_ref[0]
    oh = oh_ref[0]
    wt = wt_ref[...]
    wl_col = jnp.sum(wt * oh, axis=1, keepdims=True)
    pooled = jnp.sum(xb, axis=1, keepdims=True) * (1.0 / HW)
    y_ref[0] = jnp.sum(wt * pooled, axis=0, keepdims=True) + b_ref[...]
    s = xb * wl_col
    mx = jnp.max(s, axis=0, keepdims=True)
    mn = jnp.min(s, axis=0, keepdims=True)
    den = mx - mn
    r = 1.0 / den
    r = r * (2.0 - den * r)
    r = r * (2.0 - den * r)
    t = jnp.sum(xb * r, axis=1, keepdims=True)
    score_ref[0] = wl_col * t


def _make_sc_mask(B, C, drop):
    info = plsc.get_sparse_core_info()
    nw = info.num_cores * info.num_subcores
    rows_per = B // nw
    nslice = C // 16
    mesh = plsc.VectorSubcoreMesh(core_axis_name="c", subcore_axis_name="s")

    @functools.partial(
        pl.kernel, mesh=mesh,
        out_type=jax.ShapeDtypeStruct((B, C), jnp.float32),
        scratch_types=[
            pltpu.VMEM((rows_per, C), jnp.float32),
            pltpu.VMEM((rows_per, C), jnp.int32),
            pltpu.VMEM((rows_per, C), jnp.float32),
        ],
    )
    def mask_kernel(scores_hbm, out_hbm, rows_v, keys_v, mask_v):
        wid = lax.axis_index("s") * info.num_cores + lax.axis_index("c")
        base = wid * rows_per
        pltpu.sync_copy(scores_hbm.at[pl.ds(base, rows_per)], rows_v)
        lane = lax.iota(jnp.int32, 16)

        def _take16(x, idx):
            return lax.gather(
                x, idx[:, None],
                lax.GatherDimensionNumbers(
                    offset_dims=(), collapsed_slice_dims=(0,),
                    start_index_map=(0,)),
                (1,), mode=lax.GatherScatterMode.PROMISE_IN_BOUNDS)

        def _prefix16(v):
            for k in (1, 2, 4, 8):
                sh = _take16(v, jnp.maximum(lane - k, 0))
                v = v + jnp.where(lane >= k, sh, jnp.int32(0))
            return v

        def _splat_last(v):
            return _take16(v, jnp.full((16,), jnp.int32(15)))

        for r in range(rows_per):
            for j in range(nslice):
                v = rows_v[r, pl.ds(16 * j, 16)]
                v = jnp.where(v == 0.0, 0.0, v)
                iv = lax.bitcast_convert_type(v, jnp.int32)
                keys_v[r, pl.ds(16 * j, 16)] = (
                    iv ^ ((iv >> 31) & jnp.int32(0x7FFFFFFF)))

            lo0 = jnp.full((16,), jnp.int32(-2147483648))
            hi0 = jnp.full((16,), jnp.int32(2147483647))

            def bis(_, carry):
                lo, hi = carry
                mid = (lo >> 1) + (hi >> 1) + ((lo | hi) & 1)
                cntv = jnp.zeros((16,), jnp.int32)
                for j in range(nslice):
                    k16 = keys_v[r, pl.ds(16 * j, 16)]
                    cntv = cntv + jnp.where(k16 >= mid, jnp.int32(1),
                                            jnp.int32(0))
                cnt = _splat_last(_prefix16(cntv))
                ge = cnt >= drop
                return (jnp.where(ge, mid, lo), jnp.where(ge, hi, mid - 1))

            tau, _ = lax.fori_loop(0, 32, bis, (lo0, hi0))

            gtv = jnp.zeros((16,), jnp.int32)
            for j in range(nslice):
                k16 = keys_v[r, pl.ds(16 * j, 16)]
                gtv = gtv + jnp.where(k16 > tau, jnp.int32(1), jnp.int32(0))
            quota = drop - _splat_last(_prefix16(gtv))
            run = jnp.zeros((16,), jnp.int32)
            for j in range(nslice):
                k16 = keys_v[r, pl.ds(16 * j, 16)]
                gt = k16 > tau
                eq = k16 == tau
                eqi = jnp.where(eq, jnp.int32(1), jnp.int32(0))
                pre = _prefix16(eqi)
                cum = run + pre
                dropped = jnp.logical_or(
                    gt, jnp.logical_and(eq, cum <= quota))
                mask_v[r, pl.ds(16 * j, 16)] = jnp.where(dropped, 0.0, 1.0)
                run = run + _splat_last(pre)
        pltpu.sync_copy(mask_v, out_hbm.at[pl.ds(base, rows_per)])

    return mask_kernel


def kernel(x, labels, W, b):
    B, C, H, Wd = x.shape
    K = W.shape[0]
    HW = H * Wd
    drop = int(C * _PERCENT)
    x3 = x.reshape(B, C, HW)
    oh = (labels.astype(jnp.int32)[:, None]
          == jnp.arange(K, dtype=jnp.int32)[None, :]).astype(jnp.float32)
    oh3 = oh.reshape(B, 1, K)
    b2 = b.reshape(1, K).astype(jnp.float32)
    y, score = pl.pallas_call(
        _score_body,
        grid=(B,),
        in_specs=[
            pl.BlockSpec((1, C, HW), lambda i: (i, 0, 0)),
            pl.BlockSpec((1, 1, K), lambda i: (i, 0, 0)),
            pl.BlockSpec((C, K), lambda i: (0, 0)),
            pl.BlockSpec((1, K), lambda i: (0, 0)),
        ],
        out_specs=(
            pl.BlockSpec((1, 1, K), lambda i: (i, 0, 0)),
            pl.BlockSpec((1, C, 1), lambda i: (i, 0, 0)),
        ),
        out_shape=(
            jax.ShapeDtypeStruct((B, 1, K), jnp.float32),
            jax.ShapeDtypeStruct((B, C, 1), jnp.float32),
        ),
        compiler_params=pltpu.CompilerParams(
            dimension_semantics=("parallel",),
        ),
    )(x3, oh3, W.T, b2)
    mask = _make_sc_mask(B, C, drop)(score.reshape(B, C))
    return (y.reshape(B, K), mask.reshape(B, C, 1, 1))
